# baseline (speedup 1.0000x reference)
import functools

import jax
import jax.numpy as jnp
from jax import lax
from jax.experimental import pallas as pl
from jax.experimental.pallas import tpu as pltpu
from jax.experimental.pallas import tpu_sc as plsc

_NBUF = 4


@functools.lru_cache(maxsize=None)
def _build(V, D, B0, S):
    C = S // 2
    assert S % 2 == 0 and C <= 128
    info = plsc.get_sparse_core_info()
    NC, NS = info.num_cores, info.num_subcores
    NW = NC * NS
    B = B0 * S
    assert B % (NW * C) == 0, (B, NW, C)
    n_per_w = B // (NW * C)
    assert n_per_w % _NBUF == 0 and n_per_w >= 2 * _NBUF

    mesh = plsc.VectorSubcoreMesh(core_axis_name="c", subcore_axis_name="s")

    @functools.partial(
        pl.kernel,
        mesh=mesh,
        out_type=jax.ShapeDtypeStruct((B0, S, D), jnp.bfloat16),
        scratch_types=[
            pltpu.VMEM((n_per_w, C), jnp.int32),
            pltpu.VMEM((_NBUF, C, D), jnp.bfloat16),
        ] + [pltpu.SemaphoreType.DMA] * (2 * _NBUF),
        compiler_params=pltpu.CompilerParams(use_tc_tiling_on_sc=False),
    )
    def k(table_hbm, idx_hbm, out_hbm, idx_v, rows_v, *sems):
        gsem, wsem = sems[:_NBUF], sems[_NBUF:]
        wid = lax.axis_index("s") * NC + lax.axis_index("c")
        chunk0 = wid * n_per_w
        pltpu.sync_copy(idx_hbm.at[pl.ds(chunk0, n_per_w)], idx_v)

        def gather(b, j):
            return pltpu.make_async_copy(
                table_hbm.at[idx_v.at[j]], rows_v.at[b], gsem[b])

        def write(b, j):
            g = chunk0 + j
            return pltpu.make_async_copy(
                rows_v.at[b],
                out_hbm.at[g >> 1, pl.ds((g & 1) * C, C)],
                wsem[b])

        for b in range(_NBUF):
            gather(b, b).start()

        def outer(i, carry):
            j0 = i * _NBUF
            for b in range(_NBUF):
                j = j0 + b
                bp = (b - 1) % _NBUF

                @pl.when(jnp.logical_and(j >= 1, j + (_NBUF - 1) < n_per_w))
                def _():
                    write(bp, j - 1).wait()
                    gather(bp, j - 1 + _NBUF).start()

                gather(b, j).wait()
                write(b, j).start()
            return carry

        lax.fori_loop(0, n_per_w // _NBUF, outer, 0)
        for b in range(_NBUF):
            write(b, n_per_w - _NBUF + b).wait()

    return k


def kernel(x, weight):
    B0, S = x.shape
    V, D = weight.shape
    wb = weight.astype(jnp.bfloat16)
    idx2d = x.reshape(B0 * 2, S // 2)
    out = _build(V, D, B0, S)(wb, idx2d)
    return out.astype(jnp.float32)


# speedup vs baseline: 1.6566x; 1.6566x over previous
import functools

import jax
import jax.numpy as jnp
from jax import lax
from jax.experimental import pallas as pl
from jax.experimental.pallas import tpu as pltpu
from jax.experimental.pallas import tpu_sc as plsc

_C = 128
_NBUF = 4


@functools.lru_cache(maxsize=None)
def _build(V, B):
    info = plsc.get_sparse_core_info()
    NC, NS = info.num_cores, info.num_subcores
    NW = NC * NS
    assert B % (NW * _C) == 0, (B, NW, _C)
    n_per_w = B // (NW * _C)
    assert n_per_w % _NBUF == 0 and n_per_w >= 2 * _NBUF

    mesh = plsc.VectorSubcoreMesh(core_axis_name="c", subcore_axis_name="s")

    @functools.partial(
        pl.kernel,
        mesh=mesh,
        out_type=jax.ShapeDtypeStruct((B, 128), jnp.float32),
        scratch_types=[
            pltpu.VMEM((n_per_w, _C), jnp.int32),
            pltpu.VMEM((_NBUF, _C, 128), jnp.float32),
        ] + [pltpu.SemaphoreType.DMA] * (2 * _NBUF),
        compiler_params=pltpu.CompilerParams(use_tc_tiling_on_sc=True),
    )
    def k(table_hbm, idx_hbm, out_hbm, idx_v, rows_v, *sems):
        gsem, wsem = sems[:_NBUF], sems[_NBUF:]
        wid = lax.axis_index("s") * NC + lax.axis_index("c")
        chunk0 = pl.multiple_of(wid * n_per_w, 8)
        pltpu.sync_copy(idx_hbm.at[pl.ds(chunk0, n_per_w)], idx_v)

        def gather(b, j):
            return pltpu.make_async_copy(
                table_hbm.at[idx_v.at[j]], rows_v.at[b], gsem[b])

        def write(b, j):
            return pltpu.make_async_copy(
                rows_v.at[b], out_hbm.at[pl.ds((chunk0 + j) * _C, _C)], wsem[b])

        for b in range(_NBUF):
            gather(b, b).start()

        def outer(i, carry):
            j0 = i * _NBUF
            for b in range(_NBUF):
                j = j0 + b
                bp = (b - 1) % _NBUF

                @pl.when(jnp.logical_and(j >= 1, j + (_NBUF - 1) < n_per_w))
                def _():
                    write(bp, j - 1).wait()
                    gather(bp, j - 1 + _NBUF).start()

                gather(b, j).wait()
                write(b, j).start()
            return carry

        lax.fori_loop(0, n_per_w // _NBUF, outer, 0)
        for b in range(_NBUF):
            write(b, n_per_w - _NBUF + b).wait()

    return k


def kernel(x, weight):
    B0, S = x.shape
    V, D = weight.shape
    B = B0 * S
    wp = jnp.pad(weight, ((0, 0), (0, 128 - D)))
    idx2d = x.reshape(B // _C, _C)
    out = _build(V, B)(wp, idx2d)
    return out[:, :D].reshape(B0, S, D)
